# baseline (speedup 1.0000x reference)
import functools

import jax
import jax.numpy as jnp
from jax import lax
from jax.experimental import pallas as pl
from jax.experimental.pallas import tpu as pltpu
from jax.experimental.pallas import tpu_sc as plsc

H = 96
W = 96
NH = 16
P = 24
NB = 2 * P + 1
NC = 2
NS = 16
NW = NC * NS
IPW = H // NW
L = 16
JC = W // L
OSTRIDE = 384
TPAD = 128
KZS = 64
NBK = KZS * KZS


def _round_clip(v):
    c = jnp.minimum(jnp.maximum(v, -24.0), 24.0)
    r = c + jnp.sign(c) * 0.5
    return r.astype(jnp.int32) + P


def _rpe_body(depth_hbm, tz_hbm, txy_hbm, out_hbm,
              dep_v, zn_v, tz_v, txy_v, comb_v, kxt_v,
              obufA, obufB, semA, semB):
    wid = lax.axis_index("s") * NC + lax.axis_index("c")
    i0 = wid * IPW

    pltpu.sync_copy(depth_hbm, dep_v)
    pltpu.sync_copy(tz_hbm, tz_v)
    pltpu.sync_copy(txy_hbm, txy_v)

    def mm_body(c, carry):
        mn, mx = carry
        v = dep_v[pl.ds(c * L, L)]
        return jnp.minimum(mn, v), jnp.maximum(mx, v)

    first = dep_v[pl.ds(0, L)]
    mn, mx = lax.fori_loop(1, H * W // L, mm_body, (first, first))
    m_s = mn[0]
    x_s = mx[0]
    for k in range(1, L):
        m_s = jnp.minimum(m_s, mn[k])
        x_s = jnp.maximum(x_s, mx[k])
    r_s = (x_s - m_s) + jnp.float32(1e-8)

    def zn_body(c, _):
        zn_v[pl.ds(c * L, L)] = (dep_v[pl.ds(c * L, L)] - m_s) / r_s
        return 0

    lax.fori_loop(0, H * W // L, zn_body, 0)

    for n in range(NH):
        tzrow = [tz_v[pl.ds(n * TPAD + c * L, L)] for c in range(KZS // L)]

        def kx_body(kx, _, n=n, tzrow=tzrow):
            s = jnp.full((L,), n * TPAD, jnp.int32) + kx
            tv = plsc.load_gather(txy_v, [s])
            base = n * NBK + kx * KZS
            for c in range(KZS // L):
                comb_v[pl.ds(base + c * L, L)] = tv + tzrow[c]
            return 0

        lax.fori_loop(0, NB, kx_body, 0)

    lane = lax.iota(jnp.int32, L)
    for il in range(IPW):
        xi = jnp.full((L,), i0 + il, jnp.int32).astype(jnp.float32)
        xi = xi / jnp.float32(W - 1)
        for jc in range(JC):
            xj = (lane + jc * L).astype(jnp.float32) / jnp.float32(W - 1)
            kxt_v[pl.ds(il * W + jc * L, L)] = \
                _round_clip((xi - xj) * 24.0) * KZS

    def compute_h(h, obuf):
        for il in range(IPW):
            ia = jnp.full((L,), h * W + i0 + il, jnp.int32)
            za = plsc.load_gather(zn_v, [ia])

            @plsc.parallel_loop(0, JC, 1, unroll=JC)
            def _(jc, il=il, za=za):
                zb = zn_v[pl.ds(pl.multiple_of(h * W + jc * L, L), L)]
                kx = kxt_v[pl.ds(pl.multiple_of(il * W + jc * L, L), L)]
                ib = kx + _round_clip((za - zb) * 24.0)
                for n in range(NH):
                    v = plsc.load_gather(comb_v.at[pl.ds(n * NBK, NBK)], [ib])
                    off = pl.multiple_of(n * OSTRIDE + il * W + jc * L, L)
                    obuf[pl.ds(off, L)] = v

    def fire(h, obuf, sem):
        for n in range(NH):
            pltpu.async_copy(
                obuf.at[pl.ds(n * OSTRIDE, IPW * W)],
                out_hbm.at[pl.ds(((h * NH + n) * W + i0) * W, IPW * W)],
                sem)

    def drain(obuf, sem):
        for n in range(NH):
            pltpu.make_async_copy(
                obuf.at[pl.ds(n * OSTRIDE, IPW * W)],
                out_hbm.at[pl.ds(0, IPW * W)],
                sem).wait()

    def h_body(hh, _):
        h0 = hh * 2

        @pl.when(hh > 0)
        def _():
            drain(obufA, semA)

        compute_h(h0, obufA)
        fire(h0, obufA, semA)

        @pl.when(hh > 0)
        def _():
            drain(obufB, semB)

        compute_h(h0 + 1, obufB)
        fire(h0 + 1, obufB, semB)
        return 0

    lax.fori_loop(0, H // 2, h_body, 0)
    drain(obufA, semA)
    drain(obufB, semB)


@jax.jit
def _rpe_sc(dep_flat, tz_t, txy_t):
    mesh = plsc.VectorSubcoreMesh(core_axis_name="c", subcore_axis_name="s",
                                  num_cores=NC, num_subcores=NS)
    return pl.kernel(
        _rpe_body,
        out_type=jax.ShapeDtypeStruct((H * NH * W * W,), jnp.float32),
        mesh=mesh,
        compiler_params=pltpu.CompilerParams(needs_layout_passes=False),
        scratch_types=[
            pltpu.VMEM((H * W,), jnp.float32),
            pltpu.VMEM((H * W,), jnp.float32),
            pltpu.VMEM((NH * TPAD,), jnp.float32),
            pltpu.VMEM((NH * TPAD,), jnp.float32),
            pltpu.VMEM((NH * NBK,), jnp.float32),
            pltpu.VMEM((IPW * W,), jnp.int32),
            pltpu.VMEM((NH * OSTRIDE,), jnp.float32),
            pltpu.VMEM((NH * OSTRIDE,), jnp.float32),
            pltpu.SemaphoreType.DMA,
            pltpu.SemaphoreType.DMA,
        ],
    )(dep_flat, tz_t, txy_t)


def kernel(depth, rpe_table):
    dep_flat = depth.reshape(-1)
    tz_t = jnp.pad(rpe_table[2 * NB:3 * NB, :].T,
                   ((0, 0), (0, TPAD - NB))).reshape(-1)
    txy_t = jnp.pad((rpe_table[0:NB, :] + rpe_table[NB + P, :]).T,
                    ((0, 0), (0, TPAD - NB))).reshape(-1)
    return _rpe_sc(dep_flat, tz_t, txy_t).reshape(H, NH, W, W)


# speedup vs baseline: 107.2860x; 107.2860x over previous
import functools

import jax
import jax.numpy as jnp
from jax import lax
from jax.experimental import pallas as pl
from jax.experimental.pallas import tpu as pltpu
from jax.experimental.pallas import tpu_sc as plsc

H = 96
W = 96
NH = 16
P = 24
NB = 2 * P + 1
NC = 2
NS = 16
NW = NC * NS
IPW = H // NW
L = 16
JC = W // L
OSTRIDE = 384
TPAD = 128
KZS = 64
NBK = KZS * KZS


def _round_clip(v):
    c = jnp.minimum(jnp.maximum(v, -24.0), 24.0)
    r = c + jnp.sign(c) * 0.5
    return r.astype(jnp.int32) + P


def _rpe_body(depth_hbm, tz_hbm, txy_hbm, out_hbm,
              dep_v, zn_v, tz_v, txy_v, comb_v, kxt_v,
              obufA, obufB, semA, semB):
    wid = lax.axis_index("s") * NC + lax.axis_index("c")
    i0 = wid * IPW

    pltpu.sync_copy(depth_hbm, dep_v)
    pltpu.sync_copy(tz_hbm, tz_v)
    pltpu.sync_copy(txy_hbm, txy_v)

    def mm_body(c, carry):
        mn, mx = carry
        v = dep_v[pl.ds(c * L, L)]
        return jnp.minimum(mn, v), jnp.maximum(mx, v)

    first = dep_v[pl.ds(0, L)]
    mn, mx = lax.fori_loop(1, H * W // L, mm_body, (first, first))
    m_s = mn[0]
    x_s = mx[0]
    for k in range(1, L):
        m_s = jnp.minimum(m_s, mn[k])
        x_s = jnp.maximum(x_s, mx[k])
    r_s = (x_s - m_s) + jnp.float32(1e-8)

    @plsc.parallel_loop(0, H * W // L, 1, unroll=8)
    def _(c):
        off = pl.multiple_of(c * L, L)
        zn_v[pl.ds(off, L)] = (dep_v[pl.ds(off, L)] - m_s) / r_s

    for n in range(NH):
        tzrow = [tz_v[pl.ds(n * TPAD + c * L, L)] for c in range(KZS // L)]

        @plsc.parallel_loop(0, NB, 1, unroll=4)
        def _(kx, n=n, tzrow=tzrow):
            s = jnp.full((L,), n * TPAD, jnp.int32) + kx
            tv = plsc.load_gather(txy_v, [s])
            base = pl.multiple_of(n * NBK + kx * KZS, KZS)
            for c in range(KZS // L):
                comb_v[pl.ds(base + c * L, L)] = tv + tzrow[c]

    lane = lax.iota(jnp.int32, L)
    for il in range(IPW):
        xi = jnp.full((L,), i0 + il, jnp.int32).astype(jnp.float32)
        xi = xi / jnp.float32(W - 1)
        for jc in range(JC):
            xj = (lane + jc * L).astype(jnp.float32) / jnp.float32(W - 1)
            kxt_v[pl.ds(il * W + jc * L, L)] = \
                _round_clip((xi - xj) * 24.0) * KZS

    def compute_h(h, obuf):
        @plsc.parallel_loop(0, IPW, 1, unroll=IPW)
        def _(il):
            ia = jnp.full((L,), h * W + i0 + il, jnp.int32)
            za = plsc.load_gather(zn_v, [ia])

            @plsc.parallel_loop(0, JC, 1, unroll=JC)
            def _(jc, il=il, za=za):
                zb = zn_v[pl.ds(pl.multiple_of(h * W + jc * L, L), L)]
                kx = kxt_v[pl.ds(pl.multiple_of(il * W + jc * L, L), L)]
                ib = kx + _round_clip((za - zb) * 24.0)

                @plsc.parallel_loop(0, NH, 1, unroll=NH)
                def _(n, il=il, jc=jc, ib=ib):
                    coff = pl.multiple_of(n * NBK, NBK)
                    v = plsc.load_gather(comb_v.at[pl.ds(coff, NBK)], [ib])
                    off = pl.multiple_of(n * OSTRIDE + il * W + jc * L, L)
                    obuf[pl.ds(off, L)] = v

    def fire(h, obuf, sem):
        for n in range(NH):
            pltpu.async_copy(
                obuf.at[pl.ds(n * OSTRIDE, IPW * W)],
                out_hbm.at[pl.ds(((h * NH + n) * W + i0) * W, IPW * W)],
                sem)

    def drain(obuf, sem):
        for n in range(NH):
            pltpu.make_async_copy(
                obuf.at[pl.ds(n * OSTRIDE, IPW * W)],
                out_hbm.at[pl.ds(0, IPW * W)],
                sem).wait()

    def h_body(hh, _):
        h0 = hh * 2

        @pl.when(hh > 0)
        def _():
            drain(obufA, semA)

        compute_h(h0, obufA)
        fire(h0, obufA, semA)

        @pl.when(hh > 0)
        def _():
            drain(obufB, semB)

        compute_h(h0 + 1, obufB)
        fire(h0 + 1, obufB, semB)
        return 0

    lax.fori_loop(0, H // 2, h_body, 0)
    drain(obufA, semA)
    drain(obufB, semB)


@jax.jit
def _rpe_sc(dep_flat, tz_t, txy_t):
    mesh = plsc.VectorSubcoreMesh(core_axis_name="c", subcore_axis_name="s",
                                  num_cores=NC, num_subcores=NS)
    return pl.kernel(
        _rpe_body,
        out_type=jax.ShapeDtypeStruct((H * NH * W * W,), jnp.float32),
        mesh=mesh,
        compiler_params=pltpu.CompilerParams(needs_layout_passes=False),
        scratch_types=[
            pltpu.VMEM((H * W,), jnp.float32),
            pltpu.VMEM((H * W,), jnp.float32),
            pltpu.VMEM((NH * TPAD,), jnp.float32),
            pltpu.VMEM((NH * TPAD,), jnp.float32),
            pltpu.VMEM((NH * NBK,), jnp.float32),
            pltpu.VMEM((IPW * W,), jnp.int32),
            pltpu.VMEM((NH * OSTRIDE,), jnp.float32),
            pltpu.VMEM((NH * OSTRIDE,), jnp.float32),
            pltpu.SemaphoreType.DMA,
            pltpu.SemaphoreType.DMA,
        ],
    )(dep_flat, tz_t, txy_t)


def kernel(depth, rpe_table):
    dep_flat = depth.reshape(-1)
    tz_t = jnp.pad(rpe_table[2 * NB:3 * NB, :].T,
                   ((0, 0), (0, TPAD - NB))).reshape(-1)
    txy_t = jnp.pad((rpe_table[0:NB, :] + rpe_table[NB + P, :]).T,
                    ((0, 0), (0, TPAD - NB))).reshape(-1)
    return _rpe_sc(dep_flat, tz_t, txy_t).reshape(H, NH, W, W)


# speedup vs baseline: 188.7225x; 1.7591x over previous
import functools

import jax
import jax.numpy as jnp
from jax import lax
from jax.experimental import pallas as pl
from jax.experimental.pallas import tpu as pltpu
from jax.experimental.pallas import tpu_sc as plsc

H = 96
W = 96
NH = 16
P = 24
NB = 2 * P + 1
NC = 2
NS = 16
NW = NC * NS
IB = 24
HPG = 12
L = 16
JC = W // L
TPAD = 128
KZS = 64
NBK = KZS * KZS


def _round_clip(v):
    c = jnp.minimum(jnp.maximum(v, -24.0), 24.0)
    r = c + jnp.sign(c) * 0.5
    return r.astype(jnp.int32) + P


def _rpe_body(depth_hbm, tz_hbm, txy_hbm, out_hbm,
              zn_v, tz_v, txy_v, comb_v, kxt_v,
              obufA, obufB, semA, semB):
    wid = lax.axis_index("s") * NC + lax.axis_index("c")
    ib_id = wid & 3
    hg_id = wid >> 2
    i0 = ib_id * IB
    h0 = hg_id * HPG

    pltpu.sync_copy(depth_hbm, zn_v)
    pltpu.sync_copy(tz_hbm, tz_v)
    pltpu.sync_copy(txy_hbm, txy_v)

    def mm_body(c, carry):
        mn, mx = carry
        v = zn_v[pl.ds(c * L, L)]
        return jnp.minimum(mn, v), jnp.maximum(mx, v)

    first = zn_v[pl.ds(0, L)]
    mn, mx = lax.fori_loop(1, H * W // L, mm_body, (first, first))
    m_s = mn[0]
    x_s = mx[0]
    for k in range(1, L):
        m_s = jnp.minimum(m_s, mn[k])
        x_s = jnp.maximum(x_s, mx[k])
    r_s = (x_s - m_s) + jnp.float32(1e-8)

    @plsc.parallel_loop(0, H * W // L, 1, unroll=8)
    def _(c):
        off = pl.multiple_of(c * L, L)
        zn_v[pl.ds(off, L)] = (zn_v[pl.ds(off, L)] - m_s) / r_s

    for n in range(NH):
        tzrow = [tz_v[pl.ds(n * TPAD + c * L, L)] for c in range(KZS // L)]

        @plsc.parallel_loop(0, NB, 1, unroll=4)
        def _(kx, n=n, tzrow=tzrow):
            s = jnp.full((L,), n * TPAD, jnp.int32) + kx
            tv = plsc.load_gather(txy_v, [s])
            base = pl.multiple_of(n * NBK + kx * KZS, KZS)
            for c in range(KZS // L):
                comb_v[pl.ds(base + c * L, L)] = tv + tzrow[c]

    lane = lax.iota(jnp.int32, L)

    def kxt_body(q, _):
        xi = jnp.full((L,), i0 + q, jnp.int32).astype(jnp.float32)
        xi = xi / jnp.float32(W - 1)
        for jc in range(JC):
            xj = (lane + jc * L).astype(jnp.float32) / jnp.float32(W - 1)
            off = pl.multiple_of(q * W + jc * L, L)
            kxt_v[pl.ds(off, L)] = _round_clip((xi - xj) * 24.0) * KZS
        return 0

    lax.fori_loop(0, IB, kxt_body, 0)

    def compute_tile(h, s, obuf):
        @plsc.parallel_loop(0, 8, 1, unroll=4)
        def _(il, s=s):
            ia = jnp.full((L,), h * W + i0 + s * 8, jnp.int32) + il
            za = plsc.load_gather(zn_v, [ia])

            @plsc.parallel_loop(0, JC, 1, unroll=JC)
            def _(jc, il=il, za=za, s=s):
                zb = zn_v[pl.ds(pl.multiple_of(h * W + jc * L, L), L)]
                kxo = pl.multiple_of((s * 8 + il) * W + jc * L, L)
                kx = kxt_v[pl.ds(kxo, L)]
                ib = kx + _round_clip((za - zb) * 24.0)

                @plsc.parallel_loop(0, NH, 1, unroll=NH)
                def _(n, il=il, jc=jc, ib=ib):
                    coff = pl.multiple_of(n * NBK, NBK)
                    v = plsc.load_gather(comb_v.at[pl.ds(coff, NBK)], [ib])
                    obuf[n, il, pl.ds(jc * L, L)] = v

    def fire(h, s, obuf, sem):
        ioff = pl.multiple_of(i0 + s * 8, 8)
        pltpu.async_copy(obuf, out_hbm.at[h, :, pl.ds(ioff, 8), :], sem)

    def drain(obuf, sem):
        pltpu.make_async_copy(obuf, out_hbm.at[0, :, pl.ds(0, 8), :],
                              sem).wait()

    def h_body(hl, _):
        h = h0 + hl
        for s in range(3):
            obuf, sem = (obufB, semB) if s == 1 else (obufA, semA)
            if s == 2:
                drain(obufA, semA)
            else:
                @pl.when(hl > 0)
                def _(obuf=obuf, sem=sem):
                    drain(obuf, sem)
            compute_tile(h, s, obuf)
            fire(h, s, obuf, sem)
        return 0

    lax.fori_loop(0, HPG, h_body, 0)
    drain(obufA, semA)
    drain(obufB, semB)


@jax.jit
def _rpe_sc(dep_flat, tz_t, txy_t):
    mesh = plsc.VectorSubcoreMesh(core_axis_name="c", subcore_axis_name="s",
                                  num_cores=NC, num_subcores=NS)
    return pl.kernel(
        _rpe_body,
        out_type=jax.ShapeDtypeStruct((H, NH, W, W), jnp.float32),
        mesh=mesh,
        compiler_params=pltpu.CompilerParams(needs_layout_passes=False),
        scratch_types=[
            pltpu.VMEM((H * W,), jnp.float32),
            pltpu.VMEM((NH * TPAD,), jnp.float32),
            pltpu.VMEM((NH * TPAD,), jnp.float32),
            pltpu.VMEM((NH * NBK,), jnp.float32),
            pltpu.VMEM((IB * W,), jnp.int32),
            pltpu.VMEM((NH, 8, W), jnp.float32),
            pltpu.VMEM((NH, 8, W), jnp.float32),
            pltpu.SemaphoreType.DMA,
            pltpu.SemaphoreType.DMA,
        ],
    )(dep_flat, tz_t, txy_t)


def kernel(depth, rpe_table):
    dep_flat = depth.reshape(-1)
    tz_t = jnp.pad(rpe_table[2 * NB:3 * NB, :].T,
                   ((0, 0), (0, TPAD - NB))).reshape(-1)
    txy_t = jnp.pad((rpe_table[0:NB, :] + rpe_table[NB + P, :]).T,
                    ((0, 0), (0, TPAD - NB))).reshape(-1)
    return _rpe_sc(dep_flat, tz_t, txy_t)


# speedup vs baseline: 188.8905x; 1.0009x over previous
import functools

import jax
import jax.numpy as jnp
from jax import lax
from jax.experimental import pallas as pl
from jax.experimental.pallas import tpu as pltpu
from jax.experimental.pallas import tpu_sc as plsc

H = 96
W = 96
NH = 16
P = 24
NB = 2 * P + 1
NC = 2
NS = 16
NW = NC * NS
IB = 24
HPG = 12
L = 16
JC = W // L
TPAD = 128
KZS = 64
NBK = KZS * KZS


def _round_clip(v):
    c = jnp.minimum(jnp.maximum(v, -24.0), 24.0)
    r = c + jnp.sign(c) * 0.5
    return r.astype(jnp.int32) + P


def _rpe_body(depth_hbm, tz_hbm, txy_hbm, out_hbm,
              zn_v, tz_v, txy_v, comb_v, kxt_v,
              obufA, obufB, semA, semB):
    wid = lax.axis_index("s") * NC + lax.axis_index("c")
    ib_id = wid & 3
    hg_id = wid >> 2
    i0 = ib_id * IB
    h0 = hg_id * HPG

    pltpu.sync_copy(depth_hbm, zn_v)
    pltpu.sync_copy(tz_hbm, tz_v)
    pltpu.sync_copy(txy_hbm, txy_v)

    def mm_body(c, carry):
        mn, mx = carry
        v = zn_v[pl.ds(c * L, L)]
        return jnp.minimum(mn, v), jnp.maximum(mx, v)

    first = zn_v[pl.ds(0, L)]
    mn, mx = lax.fori_loop(1, H * W // L, mm_body, (first, first))
    m_s = mn[0]
    x_s = mx[0]
    for k in range(1, L):
        m_s = jnp.minimum(m_s, mn[k])
        x_s = jnp.maximum(x_s, mx[k])
    r_s = (x_s - m_s) + jnp.float32(1e-8)

    @plsc.parallel_loop(0, H * W // L, 1, unroll=8)
    def _(c):
        off = pl.multiple_of(c * L, L)
        zn_v[pl.ds(off, L)] = ((zn_v[pl.ds(off, L)] - m_s) / r_s) * 24.0

    for n in range(NH):
        tzrow = [tz_v[pl.ds(n * TPAD + c * L, L)] for c in range(KZS // L)]

        @plsc.parallel_loop(0, NB, 1, unroll=4)
        def _(kx, n=n, tzrow=tzrow):
            s = jnp.full((L,), n * TPAD, jnp.int32) + kx
            tv = plsc.load_gather(txy_v, [s])
            base = pl.multiple_of(n * NBK + kx * KZS, KZS)
            for c in range(KZS // L):
                comb_v[pl.ds(base + c * L, L)] = tv + tzrow[c]

    lane = lax.iota(jnp.int32, L)

    def kxt_body(q, _):
        xi = jnp.full((L,), i0 + q, jnp.int32).astype(jnp.float32)
        xi = xi / jnp.float32(W - 1)
        for jc in range(JC):
            xj = (lane + jc * L).astype(jnp.float32) / jnp.float32(W - 1)
            off = pl.multiple_of(q * W + jc * L, L)
            kxt_v[pl.ds(off, L)] = _round_clip((xi - xj) * 24.0) * KZS
        return 0

    lax.fori_loop(0, IB, kxt_body, 0)

    def compute_tile(h, s, obuf):
        @plsc.parallel_loop(0, 8, 1, unroll=4)
        def _(il, s=s):
            ia = jnp.full((L,), h * W + i0 + s * 8, jnp.int32) + il
            za = plsc.load_gather(zn_v, [ia]) + 24.5

            @plsc.parallel_loop(0, JC, 1, unroll=JC)
            def _(jc, il=il, za=za, s=s):
                zb = zn_v[pl.ds(pl.multiple_of(h * W + jc * L, L), L)]
                kxo = pl.multiple_of((s * 8 + il) * W + jc * L, L)
                kx = kxt_v[pl.ds(kxo, L)]
                ib = kx + (za - zb).astype(jnp.int32)

                @plsc.parallel_loop(0, NH, 1, unroll=NH)
                def _(n, il=il, jc=jc, ib=ib):
                    coff = pl.multiple_of(n * NBK, NBK)
                    v = plsc.load_gather(comb_v.at[pl.ds(coff, NBK)], [ib])
                    obuf[n, il, pl.ds(jc * L, L)] = v

    def fire(h, s, obuf, sem):
        ioff = pl.multiple_of(i0 + s * 8, 8)
        pltpu.async_copy(obuf, out_hbm.at[h, :, pl.ds(ioff, 8), :], sem)

    def drain(obuf, sem):
        pltpu.make_async_copy(obuf, out_hbm.at[0, :, pl.ds(0, 8), :],
                              sem).wait()

    def h_body(hl, _):
        h = h0 + hl
        for s in range(3):
            obuf, sem = (obufB, semB) if s == 1 else (obufA, semA)
            if s == 2:
                drain(obufA, semA)
            else:
                @pl.when(hl > 0)
                def _(obuf=obuf, sem=sem):
                    drain(obuf, sem)
            compute_tile(h, s, obuf)
            fire(h, s, obuf, sem)
        return 0

    lax.fori_loop(0, HPG, h_body, 0)
    drain(obufA, semA)
    drain(obufB, semB)


@jax.jit
def _rpe_sc(dep_flat, tz_t, txy_t):
    mesh = plsc.VectorSubcoreMesh(core_axis_name="c", subcore_axis_name="s",
                                  num_cores=NC, num_subcores=NS)
    return pl.kernel(
        _rpe_body,
        out_type=jax.ShapeDtypeStruct((H, NH, W, W), jnp.float32),
        mesh=mesh,
        compiler_params=pltpu.CompilerParams(needs_layout_passes=False),
        scratch_types=[
            pltpu.VMEM((H * W,), jnp.float32),
            pltpu.VMEM((NH * TPAD,), jnp.float32),
            pltpu.VMEM((NH * TPAD,), jnp.float32),
            pltpu.VMEM((NH * NBK,), jnp.float32),
            pltpu.VMEM((IB * W,), jnp.int32),
            pltpu.VMEM((NH, 8, W), jnp.float32),
            pltpu.VMEM((NH, 8, W), jnp.float32),
            pltpu.SemaphoreType.DMA,
            pltpu.SemaphoreType.DMA,
        ],
    )(dep_flat, tz_t, txy_t)


def kernel(depth, rpe_table):
    dep_flat = depth.reshape(-1)
    tz_t = jnp.pad(rpe_table[2 * NB:3 * NB, :].T,
                   ((0, 0), (0, TPAD - NB))).reshape(-1)
    txy_t = jnp.pad((rpe_table[0:NB, :] + rpe_table[NB + P, :]).T,
                    ((0, 0), (0, TPAD - NB))).reshape(-1)
    return _rpe_sc(dep_flat, tz_t, txy_t)
